# baseline (speedup 1.0000x reference)
import functools

import jax
import jax.numpy as jnp
from jax import lax
from jax.experimental import pallas as pl
from jax.experimental.pallas import tpu as pltpu

_NEG_PAD = -1e30
_LOG2E = 1.4426950408889634


def _round_up(x, m):
    return ((x + m - 1) // m) * m


def _epilogue_block(logits, labels, tile_idx, *, n_rows, tile_n, num_class,
                    lanes, aligned):
    C = num_class
    L = lanes
    TN = logits.shape[0]

    if aligned:
        valid = labels >= 0
    else:
        row = lax.broadcasted_iota(jnp.int32, (TN, 1), 0)
        real = (tile_idx * tile_n + row) < n_rows
        valid = (labels >= 0) & real

    colf = lax.broadcasted_iota(jnp.int32, (TN, L), 1).astype(jnp.float32)
    adj = jnp.where(labels < 0, labels + C, labels)
    adjf = adj.astype(jnp.float32)
    labelsf = labels.astype(jnp.float32)

    m = jnp.max(logits, axis=1, keepdims=True)
    ms = m * _LOG2E
    se = jnp.sum(jnp.exp2(logits * _LOG2E - ms), axis=1, keepdims=True)
    lse = m + jnp.log(se)

    oh = colf == adjf
    if not aligned:
        oh = oh & real
    logit_at = jnp.sum(jnp.where(oh, logits, 0.0), axis=1, keepdims=True)
    per_row_loss = jnp.where(valid, lse - logit_at, 0.0)

    predsf = jnp.min(jnp.where(logits == m, colf, float(L)),
                     axis=1, keepdims=True)
    match = predsf == labelsf
    matchf = jnp.where(match, 1.0, 0.0)
    validf = jnp.where(valid, 1.0, 0.0)

    lane8 = lax.broadcasted_iota(jnp.int32, (TN, 8), 1)
    mm = (jnp.where(lane8 == 0, 1.0, 0.0)
          + jnp.where(lane8 == 1, matchf, 0.0)
          + jnp.where(lane8 == 2, validf, 0.0))
    cnt = lax.dot_general(
        mm, jnp.where(oh, 1.0, 0.0),
        dimension_numbers=(((0,), (0,)), ((), ())),
        preferred_element_type=jnp.float32)

    loss_sum = jnp.sum(per_row_loss)
    lane = lax.broadcasted_iota(jnp.int32, (1, L), 1)
    sub = lax.broadcasted_iota(jnp.int32, (8, L), 0)
    return cnt + jnp.where((sub == 3) & (lane == 0), loss_sum, 0.0)


def _paired_kernel(feat_ref, w_ref, b_ref, lab_a_ref, lab_b_ref, out_ref,
                   buf, *, n_rows, tile_n, num_class, lanes, tiles_per_core):
    Tc = tiles_per_core
    c = pl.program_id(0)
    j = pl.program_id(1)
    TN = tile_n
    L = lanes
    aligned = (n_rows % tile_n == 0)
    common = dict(n_rows=n_rows, tile_n=tile_n, num_class=num_class,
                  lanes=lanes, aligned=aligned)

    block_s = _epilogue_block(buf[...], lab_b_ref[...], c * Tc + 2 * j - 1,
                              **common)

    dn = (((1,), (0,)), ((), ()))
    logits_a = lax.dot_general(
        feat_ref[:TN, :], w_ref[...], dimension_numbers=dn,
        preferred_element_type=jnp.float32) + b_ref[...]
    block_d = _epilogue_block(logits_a, lab_a_ref[...], c * Tc + 2 * j,
                              **common)

    buf[...] = lax.dot_general(
        feat_ref[TN:, :], w_ref[...], dimension_numbers=dn,
        preferred_element_type=jnp.float32) + b_ref[...]

    out_ref[...] = jnp.stack([block_s, block_d]).reshape(1, 1, 2, 8, L)


def _simple_kernel(feat_ref, w_ref, b_ref, labels_ref, out_ref,
                   *, n_rows, tile_n, num_class, lanes):
    logits = lax.dot_general(
        feat_ref[...], w_ref[...], dimension_numbers=(((1,), (0,)), ((), ())),
        preferred_element_type=jnp.float32) + b_ref[...]
    block = _epilogue_block(
        logits, labels_ref[...], pl.program_id(0), n_rows=n_rows,
        tile_n=tile_n, num_class=num_class, lanes=lanes,
        aligned=(n_rows % tile_n == 0))
    out_ref[...] = block.reshape(1, 8, lanes)


def kernel(feature, weight, bias, labels):
    N, D = feature.shape
    C = weight.shape[0]
    L = max(128, _round_up(C, 128))
    TN = min(1024, _round_up(N, 8))
    num_tiles = pl.cdiv(N, TN)

    w_pad = jnp.pad(weight.T.astype(feature.dtype), ((0, 0), (0, L - C)))
    b_pad = jnp.pad(bias.astype(jnp.float32).reshape(1, C),
                    ((0, 0), (0, L - C)), constant_values=_NEG_PAD)
    labels2d = labels.astype(jnp.int32).reshape(N, 1)

    common = dict(n_rows=N, tile_n=TN, num_class=C, lanes=L)

    if False and num_tiles % 4 == 0:
        Tc = num_tiles // 2
        S = Tc // 2 + 1
        npair = num_tiles // 2
        last_t = num_tiles - 1

        raw = pl.pallas_call(
            functools.partial(_paired_kernel, tiles_per_core=Tc, **common),
            grid=(2, S),
            in_specs=[
                pl.BlockSpec((2 * TN, D),
                             lambda c, j: (jnp.minimum(c * (Tc // 2) + j,
                                                       npair - 1), 0)),
                pl.BlockSpec((D, L), lambda c, j: (0, 0)),
                pl.BlockSpec((1, L), lambda c, j: (0, 0)),
                pl.BlockSpec((TN, 1),
                             lambda c, j: (jnp.minimum(c * Tc + 2 * j,
                                                       last_t), 0)),
                pl.BlockSpec((TN, 1),
                             lambda c, j: (jnp.clip(c * Tc + 2 * j - 1, 0,
                                                    last_t), 0)),
            ],
            out_specs=pl.BlockSpec((1, 1, 2, 8, L),
                                   lambda c, j: (c, j, 0, 0, 0)),
            out_shape=jax.ShapeDtypeStruct((2, S, 2, 8, L), jnp.float32),
            scratch_shapes=[pltpu.VMEM((TN, L), jnp.float32)],
            compiler_params=pltpu.CompilerParams(
                dimension_semantics=("parallel", "arbitrary"),
                vmem_limit_bytes=36 * 1024 * 1024,
            ),
        )(feature, w_pad, b_pad, labels2d, labels2d)

        part = raw.reshape(2, 2 * S, 8, L)[:, 1:Tc + 1]
        part = jnp.sum(part, axis=(0, 1))
    else:
        part = pl.pallas_call(
            functools.partial(_simple_kernel, **common),
            grid=(num_tiles,),
            in_specs=[
                pl.BlockSpec((TN, D), lambda i: (i, 0)),
                pl.BlockSpec((D, L), lambda i: (0, 0)),
                pl.BlockSpec((1, L), lambda i: (0, 0)),
                pl.BlockSpec((TN, 1), lambda i: (i, 0)),
            ],
            out_specs=pl.BlockSpec((1, 8, L), lambda i: (i, 0, 0)),
            out_shape=jax.ShapeDtypeStruct((num_tiles, 8, L), jnp.float32),
            compiler_params=pltpu.CompilerParams(
                dimension_semantics=("parallel",),
                vmem_limit_bytes=36 * 1024 * 1024,
            ),
        )(feature, w_pad, b_pad, labels2d)
        part = jnp.sum(part, axis=0)

    total = part[0, :C]
    correct = part[1, :C]
    n_valid = jnp.sum(part[2])
    acc_sum = jnp.sum(correct)
    loss_sum = part[3, 0]

    loss = loss_sum / n_valid
    acc = acc_sum / (n_valid + 1e-10)
    cat = jnp.stack([correct, total], axis=0)
    return loss, acc, cat


# speedup vs baseline: 1.0621x; 1.0621x over previous
import functools

import jax
import jax.numpy as jnp
from jax import lax
from jax.experimental import pallas as pl
from jax.experimental.pallas import tpu as pltpu

_NEG_PAD = -1e30
_LOG2E = 1.4426950408889634


def _round_up(x, m):
    return ((x + m - 1) // m) * m


def _head_kernel(feat_ref, w_ref, b_ref, labels_ref, out_ref,
                 *, n_rows, tile_n, num_class, lanes):
    C = num_class
    L = lanes
    aligned = (n_rows % tile_n == 0)

    logits = lax.dot_general(
        feat_ref[...], w_ref[...], dimension_numbers=(((1,), (0,)), ((), ())),
        preferred_element_type=jnp.float32) + b_ref[...]
    labels = labels_ref[...]
    TN = logits.shape[0]

    if aligned:
        valid = labels >= 0
    else:
        row = lax.broadcasted_iota(jnp.int32, (TN, 1), 0)
        real = (pl.program_id(0) * tile_n + row) < n_rows
        valid = (labels >= 0) & real

    colf = lax.broadcasted_iota(jnp.int32, (TN, L), 1).astype(jnp.float32)
    adj = jnp.where(labels < 0, labels + C, labels)
    adjf = adj.astype(jnp.float32)
    labelsf = labels.astype(jnp.float32)

    m = jnp.max(logits, axis=1, keepdims=True)
    ms = m * _LOG2E
    se = jnp.sum(jnp.exp2(logits * _LOG2E - ms), axis=1, keepdims=True)
    lse = m + jnp.log(se)

    oh = colf == adjf
    if not aligned:
        oh = oh & real
    logit_at = jnp.sum(jnp.where(oh, logits, 0.0), axis=1, keepdims=True)
    per_row_loss = jnp.where(valid, lse - logit_at, 0.0)

    predsf = jnp.min(jnp.where(logits == m, colf, float(L)),
                     axis=1, keepdims=True)
    match = predsf == labelsf
    matchf = jnp.where(match, 1.0, 0.0)
    validf = jnp.where(valid, 1.0, 0.0)

    lane8 = lax.broadcasted_iota(jnp.int32, (TN, 8), 1)
    mm = (jnp.where(lane8 == 0, 1.0, 0.0)
          + jnp.where(lane8 == 1, matchf, 0.0)
          + jnp.where(lane8 == 2, validf, 0.0))
    cnt = lax.dot_general(
        mm, jnp.where(oh, 1.0, 0.0),
        dimension_numbers=(((0,), (0,)), ((), ())),
        preferred_element_type=jnp.float32)

    loss_sum = jnp.sum(per_row_loss)
    lane = lax.broadcasted_iota(jnp.int32, (1, L), 1)
    sub = lax.broadcasted_iota(jnp.int32, (8, L), 0)
    block = cnt + jnp.where((sub == 3) & (lane == 0), loss_sum, 0.0)
    out_ref[...] = block.reshape(1, 8, L)


def kernel(feature, weight, bias, labels):
    N, D = feature.shape
    C = weight.shape[0]
    L = max(128, _round_up(C, 128))
    TN = min(1024, _round_up(N, 8))
    num_tiles = pl.cdiv(N, TN)

    w_pad = jnp.pad(weight.T.astype(feature.dtype), ((0, 0), (0, L - C)))
    b_pad = jnp.pad(bias.astype(jnp.float32).reshape(1, C),
                    ((0, 0), (0, L - C)), constant_values=_NEG_PAD)
    labels2d = labels.astype(jnp.int32).reshape(N, 1)

    part = pl.pallas_call(
        functools.partial(_head_kernel, n_rows=N, tile_n=TN,
                          num_class=C, lanes=L),
        grid=(num_tiles,),
        in_specs=[
            pl.BlockSpec((TN, D), lambda i: (i, 0)),
            pl.BlockSpec((D, L), lambda i: (0, 0)),
            pl.BlockSpec((1, L), lambda i: (0, 0)),
            pl.BlockSpec((TN, 1), lambda i: (i, 0)),
        ],
        out_specs=pl.BlockSpec((1, 8, L), lambda i: (i, 0, 0)),
        out_shape=jax.ShapeDtypeStruct((num_tiles, 8, L), jnp.float32),
        compiler_params=pltpu.CompilerParams(
            dimension_semantics=("parallel",),
            vmem_limit_bytes=48 * 1024 * 1024,
        ),
    )(feature, w_pad, b_pad, labels2d)

    part = jnp.sum(part, axis=0)
    total = part[0, :C]
    correct = part[1, :C]
    n_valid = jnp.sum(part[2])
    acc_sum = jnp.sum(correct)
    loss_sum = part[3, 0]

    loss = loss_sum / n_valid
    acc = acc_sum / (n_valid + 1e-10)
    cat = jnp.stack([correct, total], axis=0)
    return loss, acc, cat
